# baseline (speedup 1.0000x reference)
import jax
import jax.numpy as jnp
from jax.experimental import pallas as pl
from jax.experimental.pallas import tpu as pltpu

N_TOKENS = 32768
INPUT_DIM = 768
NUM_EXPERTS = 8
BLK_T = 8192


def _router_body(x_ref, w_ref, b_ref, o_ref):
    xb = x_ref[...].astype(jnp.bfloat16)
    wb = w_ref[...].astype(jnp.bfloat16)
    logits = jnp.dot(xb, wb, preferred_element_type=jnp.float32) + b_ref[...]
    m = jnp.max(logits, axis=-1, keepdims=True)
    e = jnp.exp(logits - m)
    s = jnp.sum(e, axis=-1, keepdims=True)
    o_ref[...] = e / s


def kernel(x, W, b):
    b2 = b.reshape(1, NUM_EXPERTS)
    grid = (N_TOKENS // BLK_T,)
    return pl.pallas_call(
        _router_body,
        grid=grid,
        in_specs=[
            pl.BlockSpec((BLK_T, INPUT_DIM), lambda i: (i, 0)),
            pl.BlockSpec((INPUT_DIM, NUM_EXPERTS), lambda i: (0, 0)),
            pl.BlockSpec((1, NUM_EXPERTS), lambda i: (0, 0)),
        ],
        out_specs=pl.BlockSpec((BLK_T, NUM_EXPERTS), lambda i: (i, 0)),
        out_shape=jax.ShapeDtypeStruct((N_TOKENS, NUM_EXPERTS), jnp.float32),
    )(x, W, b2)


# speedup vs baseline: 1.2880x; 1.2880x over previous
import jax
import jax.numpy as jnp
from jax.experimental import pallas as pl
from jax.experimental.pallas import tpu as pltpu

N_TOKENS = 32768
INPUT_DIM = 768
NUM_EXPERTS = 8
BLK_T = 8192


def _router_body(x_ref, w_ref, b_ref, o_ref):
    xb = x_ref[...].astype(jnp.bfloat16)
    wb = w_ref[...].astype(jnp.bfloat16)
    logits = jnp.dot(xb, wb, preferred_element_type=jnp.float32) + b_ref[...]
    m = jnp.max(logits, axis=-1, keepdims=True)
    e = jnp.exp(logits - m)
    s = jnp.sum(e, axis=-1, keepdims=True)
    o_ref[...] = (e / s).T


def kernel(x, W, b):
    b2 = b.reshape(1, NUM_EXPERTS)
    grid = (N_TOKENS // BLK_T,)
    out_t = pl.pallas_call(
        _router_body,
        grid=grid,
        in_specs=[
            pl.BlockSpec((BLK_T, INPUT_DIM), lambda i: (i, 0)),
            pl.BlockSpec((INPUT_DIM, NUM_EXPERTS), lambda i: (0, 0)),
            pl.BlockSpec((1, NUM_EXPERTS), lambda i: (0, 0)),
        ],
        out_specs=pl.BlockSpec((NUM_EXPERTS, BLK_T), lambda i: (0, i)),
        out_shape=jax.ShapeDtypeStruct((NUM_EXPERTS, N_TOKENS), jnp.float32),
    )(x, W, b2)
    return out_t.T


# speedup vs baseline: 1.3234x; 1.0275x over previous
import jax
import jax.numpy as jnp
from jax.experimental import pallas as pl
from jax.experimental.pallas import tpu as pltpu

N_TOKENS = 32768
INPUT_DIM = 768
NUM_EXPERTS = 8
BLK_T = 4096


def _router_body(x_ref, w_ref, b_ref, o_ref):
    xb = x_ref[...].astype(jnp.bfloat16)
    wb = w_ref[...].astype(jnp.bfloat16)
    logits = jnp.dot(xb, wb, preferred_element_type=jnp.float32) + b_ref[...]
    m = jnp.max(logits, axis=-1, keepdims=True)
    e = jnp.exp(logits - m)
    s = jnp.sum(e, axis=-1, keepdims=True)
    o_ref[...] = (e / s).T


def kernel(x, W, b):
    b2 = b.reshape(1, NUM_EXPERTS)
    grid = (N_TOKENS // BLK_T,)
    out_t = pl.pallas_call(
        _router_body,
        grid=grid,
        in_specs=[
            pl.BlockSpec((BLK_T, INPUT_DIM), lambda i: (i, 0)),
            pl.BlockSpec((INPUT_DIM, NUM_EXPERTS), lambda i: (0, 0)),
            pl.BlockSpec((1, NUM_EXPERTS), lambda i: (0, 0)),
        ],
        out_specs=pl.BlockSpec((NUM_EXPERTS, BLK_T), lambda i: (0, i)),
        out_shape=jax.ShapeDtypeStruct((NUM_EXPERTS, N_TOKENS), jnp.float32),
    )(x, W, b2)
    return out_t.T
